# baseline (speedup 1.0000x reference)
import functools

import jax
import jax.numpy as jnp
from jax import lax
from jax.experimental import pallas as pl
from jax.experimental.pallas import tpu as pltpu
from jax.experimental.pallas import tpu_sc as plsc

_L = 16


@functools.lru_cache(maxsize=None)
def _build(B, V, D):
    info = plsc.get_sparse_core_info()
    NC, NS = info.num_cores, info.num_subcores
    NW = NC * NS
    assert B % NW == 0 and D % _L == 0
    b_per_w = B // NW
    mesh = plsc.VectorSubcoreMesh(core_axis_name="c", subcore_axis_name="s")

    nchunk = 4
    rpc = b_per_w // nchunk

    @functools.partial(
        pl.kernel,
        mesh=mesh,
        out_type=jax.ShapeDtypeStruct((B, D), jnp.float32),
        compiler_params=pltpu.CompilerParams(needs_layout_passes=False),
        scratch_types=[
            pltpu.VMEM((b_per_w,), jnp.int32),
            pltpu.VMEM((b_per_w, D), jnp.float32),
            pltpu.VMEM((b_per_w,), jnp.float32),
            pltpu.SemaphoreType.DMA,
            pltpu.SemaphoreType.DMA,
        ] + [pltpu.SemaphoreType.DMA] * nchunk,
    )
    def gather_scale(idx_hbm, table_hbm, w_hbm, out_hbm, idx_v, rows_v, w_v,
                     sem_w, sem_out, *sem_c):
        wid = lax.axis_index("s") * NC + lax.axis_index("c")
        base = wid * b_per_w
        pltpu.sync_copy(idx_hbm.at[pl.ds(base, b_per_w)], idx_v)
        cp_w = pltpu.async_copy(w_hbm.at[idx_v], w_v, sem_w)
        cp_rows = [
            pltpu.async_copy(
                table_hbm.at[idx_v.at[pl.ds(c * rpc, rpc)]],
                rows_v.at[pl.ds(c * rpc, rpc)],
                sem_c[c],
            )
            for c in range(nchunk)
        ]
        cp_w.wait()
        cp_out = []
        for c in range(nchunk):
            cp_rows[c].wait()

            def _scale_row(i, carry):
                wb = plsc.load_gather(w_v, [jnp.full((_L,), 0, jnp.int32) + i])
                for j in range(D // _L):
                    sl = pl.ds(j * _L, _L)
                    rows_v[i, sl] = rows_v[i, sl] * wb
                return carry

            lax.fori_loop(c * rpc, (c + 1) * rpc, _scale_row, 0)

            cp_out.append(
                pltpu.async_copy(
                    rows_v.at[pl.ds(c * rpc, rpc)],
                    out_hbm.at[pl.ds(base + c * rpc, rpc)],
                    sem_out,
                )
            )
        for cp in cp_out:
            cp.wait()

    return gather_scale


def kernel(indices, ent_embeddings, linear_w):
    B, = indices.shape
    V, D = ent_embeddings.shape
    return _build(B, V, D)(indices, ent_embeddings,
                           jnp.squeeze(linear_w, axis=1))


# speedup vs baseline: 1.0036x; 1.0036x over previous
import functools

import jax
import jax.numpy as jnp
from jax import lax
from jax.experimental import pallas as pl
from jax.experimental.pallas import tpu as pltpu
from jax.experimental.pallas import tpu_sc as plsc

_L = 16


@functools.lru_cache(maxsize=None)
def _build(B, V, D):
    info = plsc.get_sparse_core_info()
    NC, NS = info.num_cores, info.num_subcores
    NW = NC * NS
    assert B % NW == 0 and D % _L == 0
    b_per_w = B // NW
    mesh = plsc.VectorSubcoreMesh(core_axis_name="c", subcore_axis_name="s")

    nchunk = 8
    rpc = b_per_w // nchunk

    @functools.partial(
        pl.kernel,
        mesh=mesh,
        out_type=jax.ShapeDtypeStruct((B, D), jnp.float32),
        compiler_params=pltpu.CompilerParams(needs_layout_passes=False),
        scratch_types=[
            pltpu.VMEM((b_per_w,), jnp.int32),
            pltpu.VMEM((b_per_w, D), jnp.float32),
            pltpu.VMEM((b_per_w,), jnp.float32),
            pltpu.SemaphoreType.DMA,
            pltpu.SemaphoreType.DMA,
        ] + [pltpu.SemaphoreType.DMA] * nchunk,
    )
    def gather_scale(idx_hbm, table_hbm, w_hbm, out_hbm, idx_v, rows_v, w_v,
                     sem_w, sem_out, *sem_c):
        wid = lax.axis_index("s") * NC + lax.axis_index("c")
        base = wid * b_per_w
        pltpu.sync_copy(idx_hbm.at[pl.ds(base, b_per_w)], idx_v)
        cp_w = pltpu.async_copy(w_hbm.at[idx_v], w_v, sem_w)
        cp_rows = [
            pltpu.async_copy(
                table_hbm.at[idx_v.at[pl.ds(c * rpc, rpc)]],
                rows_v.at[pl.ds(c * rpc, rpc)],
                sem_c[c],
            )
            for c in range(nchunk)
        ]
        cp_w.wait()
        cp_out = []
        for c in range(nchunk):
            cp_rows[c].wait()

            @plsc.parallel_loop(c * rpc, (c + 1) * rpc, unroll=1)
            def _scale_row(i):
                wb = plsc.load_gather(w_v, [jnp.full((_L,), 0, jnp.int32) + i])
                for j in range(D // _L):
                    sl = pl.ds(j * _L, _L)
                    rows_v[i, sl] = rows_v[i, sl] * wb

            cp_out.append(
                pltpu.async_copy(
                    rows_v.at[pl.ds(c * rpc, rpc)],
                    out_hbm.at[pl.ds(base + c * rpc, rpc)],
                    sem_out,
                )
            )
        for cp in cp_out:
            cp.wait()

    return gather_scale


def kernel(indices, ent_embeddings, linear_w):
    B, = indices.shape
    V, D = ent_embeddings.shape
    return _build(B, V, D)(indices, ent_embeddings,
                           jnp.squeeze(linear_w, axis=1))


# speedup vs baseline: 1.0270x; 1.0233x over previous
import functools

import jax
import jax.numpy as jnp
from jax import lax
from jax.experimental import pallas as pl
from jax.experimental.pallas import tpu as pltpu
from jax.experimental.pallas import tpu_sc as plsc

_L = 16


@functools.lru_cache(maxsize=None)
def _build(B, V, D):
    info = plsc.get_sparse_core_info()
    NC, NS = info.num_cores, info.num_subcores
    NW = NC * NS
    assert B % NW == 0 and D % _L == 0
    b_per_w = B // NW
    mesh = plsc.VectorSubcoreMesh(core_axis_name="c", subcore_axis_name="s")

    nchunk = 2
    rpc = b_per_w // nchunk

    @functools.partial(
        pl.kernel,
        mesh=mesh,
        out_type=jax.ShapeDtypeStruct((B, D), jnp.float32),
        compiler_params=pltpu.CompilerParams(needs_layout_passes=False),
        scratch_types=[
            pltpu.VMEM((b_per_w,), jnp.int32),
            pltpu.VMEM((b_per_w, D), jnp.float32),
            pltpu.VMEM((b_per_w,), jnp.float32),
            pltpu.SemaphoreType.DMA,
            pltpu.SemaphoreType.DMA,
        ] + [pltpu.SemaphoreType.DMA] * nchunk,
    )
    def gather_scale(idx_hbm, table_hbm, w_hbm, out_hbm, idx_v, rows_v, w_v,
                     sem_w, sem_out, *sem_c):
        wid = lax.axis_index("s") * NC + lax.axis_index("c")
        base = wid * b_per_w
        pltpu.sync_copy(idx_hbm.at[pl.ds(base, b_per_w)], idx_v)
        cp_w = pltpu.async_copy(w_hbm.at[idx_v], w_v, sem_w)
        cp_rows = [
            pltpu.async_copy(
                table_hbm.at[idx_v.at[pl.ds(c * rpc, rpc)]],
                rows_v.at[pl.ds(c * rpc, rpc)],
                sem_c[c],
            )
            for c in range(nchunk)
        ]
        cp_w.wait()
        cp_out = []
        for c in range(nchunk):
            cp_rows[c].wait()

            @plsc.parallel_loop(c * rpc, (c + 1) * rpc, unroll=1)
            def _scale_row(i):
                wb = plsc.load_gather(w_v, [jnp.full((_L,), 0, jnp.int32) + i])
                for j in range(D // _L):
                    sl = pl.ds(j * _L, _L)
                    rows_v[i, sl] = rows_v[i, sl] * wb

            cp_out.append(
                pltpu.async_copy(
                    rows_v.at[pl.ds(c * rpc, rpc)],
                    out_hbm.at[pl.ds(base + c * rpc, rpc)],
                    sem_out,
                )
            )
        for cp in cp_out:
            cp.wait()

    return gather_scale


def kernel(indices, ent_embeddings, linear_w):
    B, = indices.shape
    V, D = ent_embeddings.shape
    return _build(B, V, D)(indices, ent_embeddings,
                           jnp.squeeze(linear_w, axis=1))


# speedup vs baseline: 1.0316x; 1.0044x over previous
import functools

import jax
import jax.numpy as jnp
from jax import lax
from jax.experimental import pallas as pl
from jax.experimental.pallas import tpu as pltpu
from jax.experimental.pallas import tpu_sc as plsc

_L = 16


@functools.lru_cache(maxsize=None)
def _build(B, V, D):
    info = plsc.get_sparse_core_info()
    NC, NS = info.num_cores, info.num_subcores
    NW = NC * NS
    assert B % NW == 0 and D % _L == 0
    b_per_w = B // NW
    mesh = plsc.VectorSubcoreMesh(core_axis_name="c", subcore_axis_name="s")

    nchunk = 4
    rpc = b_per_w // nchunk

    @functools.partial(
        pl.kernel,
        mesh=mesh,
        out_type=jax.ShapeDtypeStruct((B, D), jnp.float32),
        compiler_params=pltpu.CompilerParams(needs_layout_passes=False),
        scratch_types=[
            pltpu.VMEM((b_per_w,), jnp.int32),
            pltpu.VMEM((b_per_w, D), jnp.float32),
            pltpu.VMEM((b_per_w,), jnp.float32),
            pltpu.SemaphoreType.DMA,
            pltpu.SemaphoreType.DMA,
        ] + [pltpu.SemaphoreType.DMA] * nchunk,
    )
    def gather_scale(idx_hbm, table_hbm, w_hbm, out_hbm, idx_v, rows_v, w_v,
                     sem_w, sem_out, *sem_c):
        wid = lax.axis_index("s") * NC + lax.axis_index("c")
        base = wid * b_per_w
        pltpu.sync_copy(idx_hbm.at[pl.ds(base, b_per_w)], idx_v)
        cp_w = pltpu.async_copy(w_hbm.at[idx_v], w_v, sem_w)
        cp_rows = [
            pltpu.async_copy(
                table_hbm.at[idx_v.at[pl.ds(c * rpc, rpc)]],
                rows_v.at[pl.ds(c * rpc, rpc)],
                sem_c[c],
            )
            for c in range(nchunk)
        ]
        cp_w.wait()
        cp_out = []
        for c in range(nchunk):
            cp_rows[c].wait()

            @plsc.parallel_loop(c * rpc, (c + 1) * rpc, unroll=1)
            def _scale_row(i):
                wb = plsc.load_gather(w_v, [jnp.full((_L,), i, jnp.int32)])
                for j in range(D // _L):
                    sl = pl.ds(j * _L, _L)
                    rows_v[i, sl] = rows_v[i, sl] * wb

            cp_out.append(
                pltpu.async_copy(
                    rows_v.at[pl.ds(c * rpc, rpc)],
                    out_hbm.at[pl.ds(base + c * rpc, rpc)],
                    sem_out,
                )
            )
        for cp in cp_out:
            cp.wait()

    return gather_scale


def kernel(indices, ent_embeddings, linear_w):
    B, = indices.shape
    V, D = ent_embeddings.shape
    return _build(B, V, D)(indices, ent_embeddings,
                           jnp.squeeze(linear_w, axis=1))
